# baseline (speedup 1.0000x reference)
import functools

import jax
import jax.numpy as jnp
from jax import lax
from jax.experimental import pallas as pl
from jax.experimental.pallas import tpu as pltpu
from jax.experimental.pallas import tpu_sc as plsc

_ABSORB = 0.5
_K = 8
_EPS = 1e-5


def _sim_topk_kernel(f_ref, w1_ref, g_ref, be_ref, w2_ref, idx_ref, sim_scr):
    bs, n, d = f_ref.shape[1], f_ref.shape[2], f_ref.shape[3]
    f = f_ref[0]
    ff = f.reshape(bs * n, d)
    p = jnp.dot(ff, w1_ref[:d], preferred_element_type=jnp.float32)
    q = jnp.dot(ff, w1_ref[d:], preferred_element_type=jnp.float32)

    ex = jnp.mean(p, axis=0)
    ey = jnp.mean(q, axis=0)
    ex2 = jnp.mean(p * p, axis=0)
    ey2 = jnp.mean(q * q, axis=0)
    pb = jnp.mean(p.reshape(bs, n, d), axis=1)
    qb = jnp.mean(q.reshape(bs, n, d), axis=1)
    exy = jnp.mean(pb * qb, axis=0)
    mu_xy = ex + ey
    var = ex2 + ey2 + 2.0 * exy - mu_xy * mu_xy

    alpha = g_ref[0] * jax.lax.rsqrt(var + _EPS)
    beta = be_ref[0] - alpha * mu_xy
    w2 = w2_ref[0]

    at = alpha[None, None, :] * p.reshape(bs, n, d) + beta[None, None, :]
    bt = alpha[None, None, :] * q.reshape(bs, n, d)

    row_i = jax.lax.broadcasted_iota(jnp.int32, (n, n), 0)
    col_j = jax.lax.broadcasted_iota(jnp.int32, (n, n), 1)
    col_k = jax.lax.broadcasted_iota(jnp.int32, (n, _K), 1)

    ti = 16
    for b in range(bs):
        for it in range(n // ti):
            t = at[b, it * ti:(it + 1) * ti][:, None, :] + bt[b][None, :, :]
            t = jnp.maximum(t, 0.0)
            sim_scr[it * ti:(it + 1) * ti, :] = jnp.sum(t * w2[None, None, :],
                                                        axis=-1)
        work = jnp.where(row_i == col_j, -1e9, sim_scr[...])
        cols = jnp.zeros((n, _K), jnp.int32)
        for k in range(_K):
            mx = jnp.max(work, axis=1, keepdims=True)
            eq = work == mx
            selj = jnp.min(jnp.where(eq, col_j, n), axis=1, keepdims=True)
            onehot = col_j == selj
            cols = jnp.where(col_k == k, selj, cols)
            work = jnp.where(onehot, -jnp.inf, work)
        idx_ref[0, b] = cols


def _sc_gather_kernel(f_hbm, idx_hbm, out_hbm, idx_v, idx16_v, rows_v,
                      own_v, out_v, sem):
    c = lax.axis_index("c")
    s = lax.axis_index("s")
    wid = s * 2 + c
    base_row = wid * 32
    blk_base = (base_row // 128) * 128
    pltpu.sync_copy(idx_hbm.at[pl.ds(base_row * _K, 32 * _K)], idx_v)
    pltpu.sync_copy(f_hbm.at[pl.ds(base_row, 32)], own_v)

    def pair(p, _):
        idx16_v[...] = idx_v[pl.ds(p * 16, 16)] + blk_base
        pltpu.async_copy(f_hbm.at[idx16_v], rows_v, sem).wait()
        for cc in range(16):
            sl = pl.ds(cc * 16, 16)
            acc_a = rows_v[0, sl]
            acc_b = rows_v[8, sl]
            for k in range(1, 8):
                acc_a = acc_a + rows_v[k, sl]
                acc_b = acc_b + rows_v[8 + k, sl]
            out_v[2 * p, sl] = 0.5 * own_v[2 * p, sl] + 0.0625 * acc_a
            out_v[2 * p + 1, sl] = 0.5 * own_v[2 * p + 1, sl] + 0.0625 * acc_b
        return 0

    lax.fori_loop(0, 16, pair, 0)
    pltpu.sync_copy(out_v, out_hbm.at[pl.ds(base_row, 32)])


def _cls_kernel(f1_ref, f2_ref, w1_ref, b1_ref, g_ref, be_ref, w2_ref, b2_ref,
                out_ref, h_scr, stats_scr, *, m_rows):
    p = pl.program_id(0)
    b = pl.program_id(1)
    it = pl.program_id(2)
    ti, n = out_ref.shape[2], out_ref.shape[3]
    d = f2_ref.shape[2]

    @pl.when(p == 0)
    def _():
        f1t = f1_ref[0]
        f2b = f2_ref[0]
        diff = jnp.abs(f1t[:, None, :] - f2b[None, :, :])
        h = jnp.dot(diff.reshape(ti * n, d).astype(jnp.bfloat16),
                    w1_ref[...].astype(jnp.bfloat16),
                    preferred_element_type=jnp.float32) + b1_ref[0][None, :]

        @pl.when(jnp.logical_and(b == 0, it == 0))
        def _():
            stats_scr[...] = jnp.zeros_like(stats_scr)

        stats_scr[0:1, :] += jnp.sum(h, axis=0)[None, :]
        stats_scr[1:2, :] += jnp.sum(h * h, axis=0)[None, :]
        h_scr[b, pl.ds(it * ti, ti)] = h.reshape(ti, n, d).astype(jnp.bfloat16)

    @pl.when(p == 1)
    def _():
        mean = stats_scr[0] * (1.0 / m_rows)
        var = stats_scr[1] * (1.0 / m_rows) - mean * mean
        alpha = g_ref[0] * jax.lax.rsqrt(var + _EPS)
        beta = be_ref[0] - alpha * mean
        h = h_scr[b, pl.ds(it * ti, ti)].astype(jnp.float32)
        t = jnp.maximum(alpha[None, None, :] * h + beta[None, None, :], 0.0)
        sc = jnp.sum(t * w2_ref[0][None, None, :], axis=-1) + b2_ref[0, 0]
        out_ref[0, 0] = jax.nn.sigmoid(sc)


def kernel(feat1, feat2, sim_w1, sim_b1, sim_g1, sim_be1, sim_w2, sim_b2,
           cls_w1, cls_b1, cls_g1, cls_be1, cls_w2, cls_b2):
    bs, n, d = feat1.shape
    f = jnp.stack([feat1, feat2])
    row = lambda v: v.reshape(1, -1)

    idx = pl.pallas_call(
        _sim_topk_kernel,
        grid=(2,),
        in_specs=[
            pl.BlockSpec((1, bs, n, d), lambda i: (i, 0, 0, 0)),
            pl.BlockSpec((2 * d, d), lambda i: (0, 0)),
            pl.BlockSpec((1, d), lambda i: (0, 0)),
            pl.BlockSpec((1, d), lambda i: (0, 0)),
            pl.BlockSpec((1, d), lambda i: (0, 0)),
        ],
        out_specs=pl.BlockSpec((1, bs, n, _K), lambda i: (i, 0, 0, 0)),
        out_shape=jax.ShapeDtypeStruct((2, bs, n, _K), jnp.int32),
        scratch_shapes=[pltpu.VMEM((n, n), jnp.float32)],
    )(f, sim_w1, row(sim_g1), row(sim_be1), row(sim_w2))

    ftab = f.reshape(2 * bs * n, d)
    idx_flat = idx.reshape(-1)

    mesh = plsc.VectorSubcoreMesh(core_axis_name="c", subcore_axis_name="s")
    sc_gather = functools.partial(
        pl.kernel, mesh=mesh,
        out_type=jax.ShapeDtypeStruct((2 * bs * n, d), jnp.float32),
        scratch_types=[
            pltpu.VMEM((32 * _K,), jnp.int32),
            pltpu.VMEM((16,), jnp.int32),
            pltpu.VMEM((16, d), jnp.float32),
            pltpu.VMEM((32, d), jnp.float32),
            pltpu.VMEM((32, d), jnp.float32),
            pltpu.SemaphoreType.DMA,
        ],
    )(_sc_gather_kernel)
    fo = sc_gather(ftab, idx_flat)

    f1o = fo[:bs * n].reshape(bs, n, d)
    f2o = fo[bs * n:].reshape(bs, n, d)

    ti = 64
    nt = n // ti
    b2_full = jnp.broadcast_to(cls_b2.reshape(1, 1), (1, d))
    score = pl.pallas_call(
        functools.partial(_cls_kernel, m_rows=float(bs * n * n)),
        grid=(2, bs, nt),
        in_specs=[
            pl.BlockSpec((1, ti, d), lambda p, b, it: (b, it, 0)),
            pl.BlockSpec((1, n, d), lambda p, b, it: (b, 0, 0)),
            pl.BlockSpec((d, d), lambda p, b, it: (0, 0)),
            pl.BlockSpec((1, d), lambda p, b, it: (0, 0)),
            pl.BlockSpec((1, d), lambda p, b, it: (0, 0)),
            pl.BlockSpec((1, d), lambda p, b, it: (0, 0)),
            pl.BlockSpec((1, d), lambda p, b, it: (0, 0)),
            pl.BlockSpec((1, d), lambda p, b, it: (0, 0)),
        ],
        out_specs=pl.BlockSpec((1, 1, ti, n), lambda p, b, it: (p, b, it, 0)),
        out_shape=jax.ShapeDtypeStruct((2, bs, n, n), jnp.float32),
        scratch_shapes=[
            pltpu.VMEM((bs, n, n, d), jnp.bfloat16),
            pltpu.VMEM((8, d), jnp.float32),
        ],
    )(f1o, f2o, cls_w1, row(cls_b1), row(cls_g1), row(cls_be1),
      row(cls_w2), b2_full)

    return score[1]


# speedup vs baseline: 1.6458x; 1.6458x over previous
import functools

import jax
import jax.numpy as jnp
from jax.experimental import pallas as pl
from jax.experimental.pallas import tpu as pltpu

_ABSORB = 0.5
_K = 8
_EPS = 1e-5


def _absorb(f, w1_ref, b1_ref, g_ref, be_ref, w2_ref, fo_scr, sim_scr, dst):
    bs, n, d = f.shape
    ff = f.reshape(bs * n, d)
    p = jnp.dot(ff, w1_ref[:d], preferred_element_type=jnp.float32)
    q = jnp.dot(ff, w1_ref[d:], preferred_element_type=jnp.float32)

    ex = jnp.mean(p, axis=0)
    ey = jnp.mean(q, axis=0)
    ex2 = jnp.mean(p * p, axis=0)
    ey2 = jnp.mean(q * q, axis=0)
    pb = jnp.mean(p.reshape(bs, n, d), axis=1)
    qb = jnp.mean(q.reshape(bs, n, d), axis=1)
    exy = jnp.mean(pb * qb, axis=0)
    mu_xy = ex + ey
    var = ex2 + ey2 + 2.0 * exy - mu_xy * mu_xy

    alpha = g_ref[0] * jax.lax.rsqrt(var + _EPS)
    beta = be_ref[0] - alpha * mu_xy
    w2 = w2_ref[0]

    at = alpha[None, None, :] * p.reshape(bs, n, d) + beta[None, None, :]
    bt = alpha[None, None, :] * q.reshape(bs, n, d)

    row_i = jax.lax.broadcasted_iota(jnp.int32, (n, n), 0)
    col_j = jax.lax.broadcasted_iota(jnp.int32, (n, n), 1)

    ti = 16
    for b in range(bs):
        for it in range(n // ti):
            t = at[b, it * ti:(it + 1) * ti][:, None, :] + bt[b][None, :, :]
            t = jnp.maximum(t, 0.0)
            sim_scr[it * ti:(it + 1) * ti, :] = jnp.sum(t * w2[None, None, :],
                                                        axis=-1)
        work = jnp.where(row_i == col_j, -1e9, sim_scr[...])
        mask = jnp.zeros((n, n), jnp.float32)
        for _ in range(_K):
            mx = jnp.max(work, axis=1, keepdims=True)
            eq = work == mx
            selj = jnp.min(jnp.where(eq, col_j, n), axis=1, keepdims=True)
            onehot = col_j == selj
            mask = mask + onehot.astype(jnp.float32)
            work = jnp.where(onehot, -jnp.inf, work)
        nei = jnp.dot(mask, f[b], preferred_element_type=jnp.float32) * (1.0 / _K)
        fo_scr[dst, b] = (1.0 - _ABSORB) * f[b] + _ABSORB * nei


def _gm_kernel(f1_ref, f2_ref, sw1_ref, sg_ref, sbe_ref, sw2_ref,
               cw1_ref, cb1_ref, cg_ref, cbe_ref, cw2c_ref, cb2_ref,
               out_ref, fo_scr, h_scr, stats_scr, sim_scr, *, nt, ti):
    s = pl.program_id(0)
    bs, n, d = f1_ref.shape
    m_rows = float(bs * n * n)
    p0_end = 2 + bs * nt

    @pl.when(s == 0)
    def _():
        _absorb(f1_ref[...], sw1_ref, None, sg_ref, sbe_ref, sw2_ref,
                fo_scr, sim_scr, 0)

    @pl.when(s == 1)
    def _():
        _absorb(f2_ref[...], sw1_ref, None, sg_ref, sbe_ref, sw2_ref,
                fo_scr, sim_scr, 1)

    @pl.when(jnp.logical_and(s >= 2, s < p0_end))
    def _():
        idx = s - 2
        b = idx // nt
        it = idx % nt
        f1t = fo_scr[0, b, pl.ds(it * ti, ti)]
        f2b = fo_scr[1, b]
        diff = jnp.abs(f1t[:, None, :] - f2b[None, :, :])
        h = jnp.dot(diff.reshape(ti * n, d).astype(jnp.bfloat16),
                    cw1_ref[...].astype(jnp.bfloat16),
                    preferred_element_type=jnp.float32) + cb1_ref[0][None, :]

        @pl.when(s == 2)
        def _():
            stats_scr[...] = jnp.zeros_like(stats_scr)

        stats_scr[0:1, :] += jnp.sum(h, axis=0)[None, :]
        stats_scr[1:2, :] += jnp.sum(h * h, axis=0)[None, :]
        h_scr[b, pl.ds(it * ti, ti)] = h.reshape(ti, n, d).astype(jnp.bfloat16)

    @pl.when(s >= p0_end)
    def _():
        idx = s - p0_end
        b = idx // nt
        it = idx % nt
        mean = stats_scr[0] * (1.0 / m_rows)
        var = stats_scr[1] * (1.0 / m_rows) - mean * mean
        alpha = (cg_ref[0] * jax.lax.rsqrt(var + _EPS)).astype(jnp.bfloat16)
        beta = (cbe_ref[0] - cg_ref[0] * jax.lax.rsqrt(var + _EPS) * mean
                ).astype(jnp.bfloat16)
        h = h_scr[b, pl.ds(it * ti, ti)]
        t = jnp.maximum(alpha[None, None, :] * h + beta[None, None, :],
                        jnp.bfloat16(0.0))
        sc = jnp.dot(
            t.reshape(ti * n, d), cw2c_ref[...].astype(jnp.bfloat16),
            preferred_element_type=jnp.float32).reshape(ti, n) + cb2_ref[0, 0]
        out_ref[0, 0] = jax.nn.sigmoid(sc)


def kernel(feat1, feat2, sim_w1, sim_b1, sim_g1, sim_be1, sim_w2, sim_b2,
           cls_w1, cls_b1, cls_g1, cls_be1, cls_w2, cls_b2):
    bs, n, d = feat1.shape
    ti = 64
    nt = n // ti
    p0_end = 2 + bs * nt
    steps = 2 + 2 * bs * nt
    row = lambda v: v.reshape(1, -1)
    b2_full = jnp.broadcast_to(cls_b2.reshape(1, 1), (1, d))

    const = lambda shape: pl.BlockSpec(shape, lambda s: (0,) * len(shape))

    def out_map(s):
        p = jnp.where(s >= p0_end, 1, 0)
        idx = jnp.maximum(s - p0_end, 0)
        return (p, idx // nt, idx % nt, 0)

    score = pl.pallas_call(
        functools.partial(_gm_kernel, nt=nt, ti=ti),
        grid=(steps,),
        in_specs=[
            const((bs, n, d)),
            const((bs, n, d)),
            const((2 * d, d)),
            const((1, d)),
            const((1, d)),
            const((1, d)),
            const((d, d)),
            const((1, d)),
            const((1, d)),
            const((1, d)),
            const((d, 1)),
            const((1, d)),
        ],
        out_specs=pl.BlockSpec((1, 1, ti, n), out_map),
        out_shape=jax.ShapeDtypeStruct((2, bs, n, n), jnp.float32),
        scratch_shapes=[
            pltpu.VMEM((2, bs, n, d), jnp.float32),
            pltpu.VMEM((bs, n, n, d), jnp.bfloat16),
            pltpu.VMEM((8, d), jnp.float32),
            pltpu.VMEM((n, n), jnp.float32),
        ],
    )(feat1, feat2, sim_w1, row(sim_g1), row(sim_be1), row(sim_w2),
      cls_w1, row(cls_b1), row(cls_g1), row(cls_be1), cls_w2, b2_full)

    return score[1]
